# baseline (speedup 1.0000x reference)
import functools

import jax
import jax.numpy as jnp
from jax import lax
from jax.experimental import pallas as pl
from jax.experimental.pallas import tpu as pltpu
from jax.experimental.pallas import tpu_sc as plsc

N_NODES = 10000
D = 256
E = 160000

NC = 2
NS = 16
NW = NC * NS

EPW = E // NW
B = 64
NFULL = EPW // B
TAIL = EPW - NFULL * B
L = 16
NFC = D // L
PAD = NFULL * B + 2 * L


def _proj_body(z_ref, w_ref, b_ref, o_ref):
    o_ref[...] = (
        lax.dot_general(
            z_ref[...].astype(jnp.bfloat16), w_ref[...].astype(jnp.bfloat16),
            (((1,), (1,)), ((), ())),
            preferred_element_type=jnp.float32,
        )
        + b_ref[...]
    ).astype(jnp.bfloat16)


def _project(z, W, b):
    blk = 2000
    return pl.pallas_call(
        _proj_body,
        grid=(N_NODES // blk,),
        in_specs=[
            pl.BlockSpec((blk, D), lambda i: (i, 0)),
            pl.BlockSpec((D, D), lambda i: (0, 0)),
            pl.BlockSpec((1, D), lambda i: (0, 0)),
        ],
        out_specs=pl.BlockSpec((blk, D), lambda i: (i, 0)),
        out_shape=jax.ShapeDtypeStruct((N_NODES, D), jnp.bfloat16),
    )(z, W, b.reshape(1, D))


def _decode_body(zp_hbm, ei_hbm, out_hbm,
                 idx0_v, idx1_v, outl_v,
                 srca, dsta, sema, sema_h,
                 srcb, dstb, semb, semb_h,
                 zps):
    wid = lax.axis_index("s") * NC + lax.axis_index("c")
    sid = lax.axis_index("s")
    span = wid * EPW
    lane = lax.iota(jnp.int32, L)

    rpt = N_NODES // NS
    cp_z = pltpu.make_async_copy(zp_hbm.at[pl.ds(sid * rpt, rpt)],
                                 zps.at[pl.ds(sid * rpt, rpt)], sema)
    cp_i0 = pltpu.make_async_copy(ei_hbm.at[0, pl.ds(span, EPW)],
                                  idx0_v.at[pl.ds(0, EPW)], sema)
    cp_i1 = pltpu.make_async_copy(ei_hbm.at[1, pl.ds(span, EPW)],
                                  idx1_v.at[pl.ds(0, EPW)], sema)
    cp_z.start()
    cp_i0.start()
    cp_i1.start()
    cp_z.wait()
    cp_i0.wait()
    cp_i1.wait()
    plsc.subcore_barrier()

    bufs = ((srca, dsta, sema, sema_h), (srcb, dstb, semb, semb_h))

    def start(c, buf, n):
        src_v, dst_v, sem, sem_h = buf
        i0 = idx0_v.at[pl.ds(c * B, n)]
        i1 = idx1_v.at[pl.ds(c * B, n)]
        pltpu.make_async_copy(
            zps.at[i0], src_v.at[pl.ds(0, n)], sem).start()
        pltpu.make_async_copy(
            zp_hbm.at[i1], dst_v.at[pl.ds(0, n)], sem_h).start()

    def wait_compute(c, buf, n, ngroup):
        src_v, dst_v, sem, sem_h = buf
        i0 = idx0_v.at[pl.ds(c * B, n)]
        i1 = idx1_v.at[pl.ds(c * B, n)]
        pltpu.make_async_copy(
            zps.at[i0], src_v.at[pl.ds(0, n)], sem).wait()
        pltpu.make_async_copy(
            zp_hbm.at[i1], dst_v.at[pl.ds(0, n)], sem_h).wait()

        def group_body(g, _):
            r = jnp.zeros((L,), jnp.float32)
            for el in range(L):
                e = g * L + el
                prods = []
                for i in range(D // 32):
                    p = (src_v[e, pl.ds(i * 32, 32)]
                         * dst_v[e, pl.ds(i * 32, 32)])
                    pe, po = plsc.unpack(
                        p, format=plsc.PackFormat.INTERLEAVED)
                    prods += [pe, po]
                while len(prods) > 1:
                    prods = [prods[k] + prods[k + 1]
                             for k in range(0, len(prods), 2)]
                tot = jnp.sum(prods[0])
                r = jnp.where(lane == el, tot, r)
            outl_v[pl.ds(c * B + g * L, L)] = 1.0 / (1.0 + jnp.exp(-r))
            return 0

        lax.fori_loop(0, ngroup, group_body, 0)

    start(0, bufs[0], B)

    def pair_body(j, _):
        c0 = 2 * j
        start(c0 + 1, bufs[1], B)
        wait_compute(c0, bufs[0], B, B // L)
        start(c0 + 2, bufs[0], B)
        wait_compute(c0 + 1, bufs[1], B, B // L)
        return 0

    if NFULL % 2 == 0:
        lax.fori_loop(0, NFULL // 2 - 1, pair_body, 0)
        start(NFULL - 1, bufs[1], B)
        wait_compute(NFULL - 2, bufs[0], B, B // L)
        wait_compute(NFULL - 1, bufs[1], B, B // L)
        tail_buf = bufs[0]
    else:
        lax.fori_loop(0, (NFULL - 1) // 2, pair_body, 0)
        wait_compute(NFULL - 1, bufs[0], B, B // L)
        tail_buf = bufs[1]

    start(NFULL, tail_buf, TAIL)
    wait_compute(NFULL, tail_buf, TAIL, 1)

    pltpu.sync_copy(outl_v.at[pl.ds(0, EPW)], out_hbm.at[pl.ds(span, EPW)])


_decode = functools.partial(
    pl.kernel,
    mesh=plsc.VectorSubcoreMesh(core_axis_name="c", subcore_axis_name="s"),
    out_type=jax.ShapeDtypeStruct((E,), jnp.float32),
    compiler_params=pltpu.CompilerParams(
        use_tc_tiling_on_sc=False, needs_layout_passes=False
    ),
    scratch_types=[
        pltpu.VMEM((EPW,), jnp.int32),
        pltpu.VMEM((EPW,), jnp.int32),
        pltpu.VMEM((PAD,), jnp.float32),
        pltpu.VMEM((B, D), jnp.bfloat16),
        pltpu.VMEM((B, D), jnp.bfloat16),
        pltpu.SemaphoreType.DMA,
        pltpu.SemaphoreType.DMA,
        pltpu.VMEM((B, D), jnp.bfloat16),
        pltpu.VMEM((B, D), jnp.bfloat16),
        pltpu.SemaphoreType.DMA,
        pltpu.SemaphoreType.DMA,
        pltpu.VMEM_SHARED((N_NODES, D), jnp.bfloat16),
    ],
)(_decode_body)


def kernel(z, edge_index, W, b):
    zp = _project(z, W, b)
    return _decode(zp, edge_index.astype(jnp.int32))


# speedup vs baseline: 1.0976x; 1.0976x over previous
import functools

import jax
import jax.numpy as jnp
from jax import lax
from jax.experimental import pallas as pl
from jax.experimental.pallas import tpu as pltpu
from jax.experimental.pallas import tpu_sc as plsc

N_NODES = 10000
D = 256
E = 160000

NC = 2
NS = 16
NW = NC * NS

EPW = E // NW
B = 64
NFULL = EPW // B
TAIL = EPW - NFULL * B
L = 16
NFC = D // L
PAD = NFULL * B + 2 * L


def _proj_body(z_ref, w_ref, b_ref, o_ref):
    o_ref[...] = (
        lax.dot_general(
            z_ref[...].astype(jnp.bfloat16), w_ref[...].astype(jnp.bfloat16),
            (((1,), (1,)), ((), ())),
            preferred_element_type=jnp.float32,
        )
        + b_ref[...]
    ).astype(jnp.bfloat16)


def _project(z, W, b):
    blk = 2000
    return pl.pallas_call(
        _proj_body,
        grid=(N_NODES // blk,),
        in_specs=[
            pl.BlockSpec((blk, D), lambda i: (i, 0)),
            pl.BlockSpec((D, D), lambda i: (0, 0)),
            pl.BlockSpec((1, D), lambda i: (0, 0)),
        ],
        out_specs=pl.BlockSpec((blk, D), lambda i: (i, 0)),
        out_shape=jax.ShapeDtypeStruct((N_NODES, D), jnp.bfloat16),
    )(z, W, b.reshape(1, D))


def _decode_body(zp_hbm, ei_hbm, out_hbm,
                 idx0_v, idx1_v, outl_v,
                 srca, dsta, sema, sema_h,
                 srcb, dstb, semb, semb_h,
                 zps):
    wid = lax.axis_index("s") * NC + lax.axis_index("c")
    sid = lax.axis_index("s")
    span = wid * EPW
    lane = lax.iota(jnp.int32, L)

    rpt = N_NODES // NS
    cp_z = pltpu.make_async_copy(zp_hbm.at[pl.ds(sid * rpt, rpt)],
                                 zps.at[pl.ds(sid * rpt, rpt)], sema)
    cp_i0 = pltpu.make_async_copy(ei_hbm.at[0, pl.ds(span, EPW)],
                                  idx0_v.at[pl.ds(0, EPW)], sema)
    cp_i1 = pltpu.make_async_copy(ei_hbm.at[1, pl.ds(span, EPW)],
                                  idx1_v.at[pl.ds(0, EPW)], sema)
    cp_z.start()
    cp_i0.start()
    cp_i1.start()
    cp_z.wait()
    cp_i0.wait()
    cp_i1.wait()
    plsc.subcore_barrier()

    bufs = ((srca, dsta, sema, sema_h), (srcb, dstb, semb, semb_h))

    def start(c, buf, n):
        src_v, dst_v, sem, sem_h = buf
        i0 = idx0_v.at[pl.ds(c * B, n)]
        i1 = idx1_v.at[pl.ds(c * B, n)]
        pltpu.make_async_copy(
            zps.at[i0], src_v.at[pl.ds(0, n)], sem).start()
        pltpu.make_async_copy(
            zps.at[i1], dst_v.at[pl.ds(0, n)], sem_h).start()

    def wait_compute(c, buf, n, ngroup):
        src_v, dst_v, sem, sem_h = buf
        i0 = idx0_v.at[pl.ds(c * B, n)]
        i1 = idx1_v.at[pl.ds(c * B, n)]
        pltpu.make_async_copy(
            zps.at[i0], src_v.at[pl.ds(0, n)], sem).wait()
        pltpu.make_async_copy(
            zps.at[i1], dst_v.at[pl.ds(0, n)], sem_h).wait()

        def group_body(g, _):
            r = jnp.zeros((L,), jnp.float32)
            for el in range(L):
                e = g * L + el
                prods = []
                for i in range(D // 32):
                    p = (src_v[e, pl.ds(i * 32, 32)]
                         * dst_v[e, pl.ds(i * 32, 32)])
                    pe, po = plsc.unpack(
                        p, format=plsc.PackFormat.INTERLEAVED)
                    prods += [pe, po]
                while len(prods) > 1:
                    prods = [prods[k] + prods[k + 1]
                             for k in range(0, len(prods), 2)]
                tot = jnp.sum(prods[0])
                r = jnp.where(lane == el, tot, r)
            outl_v[pl.ds(c * B + g * L, L)] = 1.0 / (1.0 + jnp.exp(-r))
            return 0

        lax.fori_loop(0, ngroup, group_body, 0)

    start(0, bufs[0], B)

    def pair_body(j, _):
        c0 = 2 * j
        start(c0 + 1, bufs[1], B)
        wait_compute(c0, bufs[0], B, B // L)
        start(c0 + 2, bufs[0], B)
        wait_compute(c0 + 1, bufs[1], B, B // L)
        return 0

    if NFULL % 2 == 0:
        lax.fori_loop(0, NFULL // 2 - 1, pair_body, 0)
        start(NFULL - 1, bufs[1], B)
        wait_compute(NFULL - 2, bufs[0], B, B // L)
        wait_compute(NFULL - 1, bufs[1], B, B // L)
        tail_buf = bufs[0]
    else:
        lax.fori_loop(0, (NFULL - 1) // 2, pair_body, 0)
        wait_compute(NFULL - 1, bufs[0], B, B // L)
        tail_buf = bufs[1]

    start(NFULL, tail_buf, TAIL)
    wait_compute(NFULL, tail_buf, TAIL, 1)

    pltpu.sync_copy(outl_v.at[pl.ds(0, EPW)], out_hbm.at[pl.ds(span, EPW)])


_decode = functools.partial(
    pl.kernel,
    mesh=plsc.VectorSubcoreMesh(core_axis_name="c", subcore_axis_name="s"),
    out_type=jax.ShapeDtypeStruct((E,), jnp.float32),
    compiler_params=pltpu.CompilerParams(
        use_tc_tiling_on_sc=False, needs_layout_passes=False
    ),
    scratch_types=[
        pltpu.VMEM((EPW,), jnp.int32),
        pltpu.VMEM((EPW,), jnp.int32),
        pltpu.VMEM((PAD,), jnp.float32),
        pltpu.VMEM((B, D), jnp.bfloat16),
        pltpu.VMEM((B, D), jnp.bfloat16),
        pltpu.SemaphoreType.DMA,
        pltpu.SemaphoreType.DMA,
        pltpu.VMEM((B, D), jnp.bfloat16),
        pltpu.VMEM((B, D), jnp.bfloat16),
        pltpu.SemaphoreType.DMA,
        pltpu.SemaphoreType.DMA,
        pltpu.VMEM_SHARED((N_NODES, D), jnp.bfloat16),
    ],
)(_decode_body)


def kernel(z, edge_index, W, b):
    zp = _project(z, W, b)
    return _decode(zp, edge_index.astype(jnp.int32))
